# baseline (speedup 1.0000x reference)
import functools

import jax
import jax.numpy as jnp
import numpy as np
from jax import lax
from jax.experimental import pallas as pl
from jax.experimental.pallas import tpu as pltpu
from jax.experimental.pallas import tpu_sc as plsc

N = 10000
E = 320000
H = 128
MID = 64
F1 = 12
PE = 16

NC = 2
NS = 16
C = 40
CPT = E // C // NS
K = 10
BPT = CPT // K
EPT = E // NS
NPAD = 10240
RPT = NPAD // NS
ZR = 128

f32 = jnp.float32


def _mm(a, w):
    return lax.dot_general(a, w, (((1,), (1,)), ((), ())),
                           preferred_element_type=f32)


def _swish(y):
    return y * jax.nn.sigmoid(y)


def _pack_pairs(y):
    a = y[:, :H // 2].astype(jnp.bfloat16).astype(f32)
    b = y[:, H // 2:].astype(jnp.bfloat16).astype(f32)
    au = lax.bitcast_convert_type(a, jnp.uint32)
    bu = lax.bitcast_convert_type(b, jnp.uint32)
    return lax.bitcast_convert_type((au >> 16) | bu, jnp.int32)


def _xlin_body(x_ref, w1, b1, w2, b2, o1, o2):
    x = x_ref[...]
    y1 = _swish(_mm(x, w1[...]) + b1[...])
    o1[...] = y1
    y2 = _mm(x, w2[...]) + b2[...]
    o2[...] = _swish(y2)


def _xlin_call(x, w1, b1, w2, b2):
    BN = 2000
    full = pl.BlockSpec((H, H), lambda i: (0, 0))
    bias = pl.BlockSpec((1, H), lambda i: (0, 0))
    return pl.pallas_call(
        _xlin_body,
        grid=(N // BN,),
        in_specs=[pl.BlockSpec((BN, H), lambda i: (i, 0)), full, bias, full, bias],
        out_specs=[pl.BlockSpec((BN, H), lambda i: (i, 0))] * 2,
        out_shape=[jax.ShapeDtypeStruct((N, H), f32)] * 2,
    )(x, w1, b1, w2, b2)


def _feat_body(fe_ref, pe_ref, wa1, wb1, wa2, wb2, o1, o2):
    m1 = _mm(wb1[...], wa1[...].T)
    o1[...] = _pack_pairs(_mm(fe_ref[...], m1))
    m2 = _mm(wb2[...], wa2[...].T)
    o2[...] = _pack_pairs(_mm(pe_ref[...], m2))


def _feat_call(feature1, pos_emb, wa1, wb1, wa2, wb2):
    BE = 4000
    return pl.pallas_call(
        _feat_body,
        grid=(E // BE,),
        in_specs=[
            pl.BlockSpec((BE, F1), lambda i: (i, 0)),
            pl.BlockSpec((BE, PE), lambda i: (i, 0)),
            pl.BlockSpec((MID, F1), lambda i: (0, 0)),
            pl.BlockSpec((H, MID), lambda i: (0, 0)),
            pl.BlockSpec((MID, PE), lambda i: (0, 0)),
            pl.BlockSpec((H, MID), lambda i: (0, 0)),
        ],
        out_specs=[pl.BlockSpec((BE, H // 2), lambda i: (i, 0))] * 2,
        out_shape=[jax.ShapeDtypeStruct((E, H // 2), jnp.int32)] * 2,
    )(feature1, pos_emb, wa1, wb1, wa2, wb2)


def _sc_body(xw, f1, f2, src3, dst3, agg1, agg2,
             shared, idx_s, idx_d, rows0, rows1, fb0, fb1, out0, out1,
             gs0, gs1, fs0, fs1, ss0, ss1):
    c = lax.axis_index("c")
    s = lax.axis_index("s")
    rows = [rows0, rows1]
    fbuf = [fb0, fb1]
    obuf = [out0, out1]
    gsem = [gs0, gs1]
    fsem = [fs0, fs1]
    ssem = [ss0, ss1]

    def zrow(i, _):
        for j in range(H // 16):
            out0[i, pl.ds(j * 16, 16)] = jnp.zeros((16,), f32)
        return 0
    lax.fori_loop(0, C, zrow, 0)
    for k in range(RPT // C):
        pltpu.sync_copy(out0, shared.at[pl.ds(s * RPT + k * C, C)])
    plsc.subcore_barrier()

    def edge_loop(f_hbm):
        def block(i, _):
            blk = s * BPT + i
            pltpu.sync_copy(src3.at[blk], idx_s)
            pltpu.sync_copy(dst3.at[blk], idx_d)

            def gat(j):
                return pltpu.async_copy(xw.at[idx_s.at[j]], rows[j % 2],
                                        gsem[j % 2])

            def fld(j):
                fbase = blk * (K * C) + j * C
                return pltpu.async_copy(
                    f_hbm.at[pl.ds(fbase, C)], fbuf[j % 2], fsem[j % 2])

            g = [None] * K
            fd = [None] * K
            sd = [None] * K
            g[0] = gat(0)
            fd[0] = fld(0)
            g[1] = gat(1)
            fd[1] = fld(1)
            for j in range(K):
                b = j % 2
                if j >= 2:
                    sd[j - 2].wait()
                g[j].wait()
                fd[j].wait()

                fb = fbuf[b]
                rw = rows[b]
                ob = obuf[b]

                @plsc.parallel_loop(0, C // 2, step=1, unroll=2)
                def _(pp):
                    for par in range(2):
                        e = 2 * pp + par
                        for q in range(H // 32):
                            xa = rw[e, pl.ds(q * 16, 16)]
                            xb = rw[e, pl.ds(H // 2 + q * 16, 16)]
                            fwv = fb[e, pl.ds(q * 16, 16)]
                            fa = lax.bitcast_convert_type(fwv << 16, f32)
                            fbb = lax.bitcast_convert_type(
                                fwv & (-65536), f32)
                            ob[e, pl.ds(q * 16, 16)] = xa * fa
                            ob[e, pl.ds(H // 2 + q * 16, 16)] = xb * fbb

                sd[j] = pltpu.async_copy(ob, shared.at[idx_d.at[j]],
                                         ssem[b], add=True)
                if j + 2 < K:
                    g[j + 2] = gat(j + 2)
                    fd[j + 2] = fld(j + 2)
            sd[K - 2].wait()
            sd[K - 1].wait()
            return 0
        lax.fori_loop(0, BPT, block, 0)

    @pl.when(c == 0)
    def _():
        edge_loop(f1)

    @pl.when(c == 1)
    def _():
        edge_loop(f2)

    plsc.subcore_barrier()

    @pl.when(c == 0)
    def _():
        pltpu.sync_copy(shared.at[pl.ds(s * RPT, RPT)],
                        agg1.at[pl.ds(s * RPT, RPT)])

    @pl.when(c == 1)
    def _():
        pltpu.sync_copy(shared.at[pl.ds(s * RPT, RPT)],
                        agg2.at[pl.ds(s * RPT, RPT)])


def _sc_call(xw, f1, f2, src2, dst2):
    mesh = plsc.VectorSubcoreMesh(core_axis_name="c", subcore_axis_name="s")
    return pl.kernel(
        _sc_body,
        out_type=[jax.ShapeDtypeStruct((NPAD, H), f32)] * 2,
        mesh=mesh,
        scratch_types=[
            pltpu.VMEM_SHARED((NPAD, H), f32),
            pltpu.VMEM((K, C), jnp.int32),
            pltpu.VMEM((K, C), jnp.int32),
            pltpu.VMEM((C, H), f32),
            pltpu.VMEM((C, H), f32),
            pltpu.VMEM((C, H // 2), jnp.int32),
            pltpu.VMEM((C, H // 2), jnp.int32),
            pltpu.VMEM((C, H), f32),
            pltpu.VMEM((C, H), f32),
        ] + [pltpu.SemaphoreType.DMA] * 6,
    )(xw, f1, f2, src2, dst2)


def _tail_body(a1, a2, x1, x2,
               wc1l, bc1l, wc1r, wl1, bl1,
               wc2l, bc2l, wc2r, wl2, bl2,
               wc0a, wc0b, bc0, wca1, bca1, wca2, bca2,
               wl0, bl0, wll1, bll1, wf, bf, out):
    xl1 = x1[...]
    h1 = _mm(a1[...], wc1l[...]) + bc1l[...] + _mm(xl1, wc1r[...])
    h1 = _swish(_mm(h1, wl1[...]) + bl1[...])
    h2 = _mm(a2[...], wc2l[...]) + bc2l[...] + _mm(xl1, wc2r[...])
    h2 = _swish(_mm(h2, wl2[...]) + bl2[...])
    h = _swish(_mm(h1, wc0a[...]) + _mm(h2, wc0b[...]) + bc0[...])
    h = _swish(_mm(h, wca1[...]) + bca1[...])
    h = _swish(_mm(h, wca2[...]) + bca2[...])
    h = h + x2[...]
    h = _swish(_mm(h, wl0[...]) + bl0[...])
    h = _swish(_mm(h, wll1[...]) + bll1[...])
    out[...] = _mm(h, wf[...]) + bf[...]


def _tail_call(a1, a2, x1, x2, weights):
    BN = 2000
    blk = pl.BlockSpec((BN, H), lambda i: (i, 0))
    full = pl.BlockSpec((H, H), lambda i: (0, 0))
    bias = pl.BlockSpec((1, H), lambda i: (0, 0))
    wspecs = []
    for w in weights:
        wspecs.append(bias if w.shape[0] == 1 else full)
    return pl.pallas_call(
        _tail_body,
        grid=(N // BN,),
        in_specs=[blk, blk, blk, blk] + wspecs,
        out_specs=blk,
        out_shape=jax.ShapeDtypeStruct((N, H), f32),
    )(a1, a2, x1, x2, *weights)


def kernel(x, feature1, pos_emb, edge_index, batch, params):
    p = params
    src2 = edge_index[0].astype(jnp.int32).reshape(NS * BPT, K, C)
    dst2 = edge_index[1].astype(jnp.int32).reshape(NS * BPT, K, C)

    def b(name):
        return p[name].reshape(1, H)

    xl1, xl2 = _xlin_call(x, p['W_lin_1'], b('b_lin_1'),
                          p['W_lin_2'], b('b_lin_2'))
    f1, f2 = _feat_call(feature1, pos_emb,
                        p['Wf1_a'], p['Wf1_b'], p['Wf2_a'], p['Wf2_b'])
    agg1, agg2 = _sc_call(xl1, f1, f2, src2, dst2)

    weights = [
        p['Wc1_l'], b('bc1_l'), p['Wc1_r'], p['W_lin1'], b('b_lin1'),
        p['Wc2_l'], b('bc2_l'), p['Wc2_r'], p['W_lin2'], b('b_lin2'),
        p['W_cat0'][:, :H], p['W_cat0'][:, H:], b('b_cat0'),
        p['W_cat1'], b('b_cat1'), p['W_cat2'], b('b_cat2'),
        p['W_l0'], b('b_l0'), p['W_l1'], b('b_l1'),
        p['W_final'], b('b_final'),
    ]
    return _tail_call(agg1, agg2, xl1, xl2, weights)


# speedup vs baseline: 1.0480x; 1.0480x over previous
import functools

import jax
import jax.numpy as jnp
import numpy as np
from jax import lax
from jax.experimental import pallas as pl
from jax.experimental.pallas import tpu as pltpu
from jax.experimental.pallas import tpu_sc as plsc

N = 10000
E = 320000
H = 128
MID = 64
F1 = 12
PE = 16

NC = 2
NS = 16
C = 40
CPT = E // C // NS
K = 10
BPT = CPT // K
EPT = E // NS
NPAD = 10240
RPT = NPAD // NS
ZR = 128

f32 = jnp.float32


def _mm(a, w):
    return lax.dot_general(a, w, (((1,), (1,)), ((), ())),
                           preferred_element_type=f32)


def _swish(y):
    return y * jax.nn.sigmoid(y)


def _pack_pairs(y):
    a = y[:, :H // 2].astype(jnp.bfloat16).astype(f32)
    b = y[:, H // 2:].astype(jnp.bfloat16).astype(f32)
    au = lax.bitcast_convert_type(a, jnp.uint32)
    bu = lax.bitcast_convert_type(b, jnp.uint32)
    return lax.bitcast_convert_type((au >> 16) | bu, jnp.int32)


def _xlin_body(x_ref, w1, b1, w2, b2, o1, o2):
    x = x_ref[...]
    y1 = _swish(_mm(x, w1[...]) + b1[...])
    o1[...] = y1
    y2 = _mm(x, w2[...]) + b2[...]
    o2[...] = _swish(y2)


def _xlin_call(x, w1, b1, w2, b2):
    BN = 2000
    full = pl.BlockSpec((H, H), lambda i: (0, 0))
    bias = pl.BlockSpec((1, H), lambda i: (0, 0))
    return pl.pallas_call(
        _xlin_body,
        grid=(N // BN,),
        in_specs=[pl.BlockSpec((BN, H), lambda i: (i, 0)), full, bias, full, bias],
        out_specs=[pl.BlockSpec((BN, H), lambda i: (i, 0))] * 2,
        out_shape=[jax.ShapeDtypeStruct((N, H), f32)] * 2,
    )(x, w1, b1, w2, b2)


def _feat_body(fe_ref, pe_ref, wa1, wb1, wa2, wb2, o1, o2):
    m1 = _mm(wb1[...], wa1[...].T)
    o1[...] = _pack_pairs(_mm(fe_ref[...], m1))
    m2 = _mm(wb2[...], wa2[...].T)
    o2[...] = _pack_pairs(_mm(pe_ref[...], m2))


def _feat_call(feature1, pos_emb, wa1, wb1, wa2, wb2):
    BE = 4000
    return pl.pallas_call(
        _feat_body,
        grid=(E // BE,),
        in_specs=[
            pl.BlockSpec((BE, F1), lambda i: (i, 0)),
            pl.BlockSpec((BE, PE), lambda i: (i, 0)),
            pl.BlockSpec((MID, F1), lambda i: (0, 0)),
            pl.BlockSpec((H, MID), lambda i: (0, 0)),
            pl.BlockSpec((MID, PE), lambda i: (0, 0)),
            pl.BlockSpec((H, MID), lambda i: (0, 0)),
        ],
        out_specs=[pl.BlockSpec((BE, H // 2), lambda i: (i, 0))] * 2,
        out_shape=[jax.ShapeDtypeStruct((E, H // 2), jnp.int32)] * 2,
    )(feature1, pos_emb, wa1, wb1, wa2, wb2)


NIDX = 5


def _sc_body(xw, f1, f2, src, dst, agg1, agg2,
             shared, rows0, rows1, fb0, fb1, out0, out1,
             sb0, sb1, sb2, sb3, sb4, db0, db1, db2, db3, db4,
             gs0, gs1, fs0, fs1, ss0, ss1,
             is0, is1, is2, is3, is4, id0, id1, id2, id3, id4):
    c = lax.axis_index("c")
    s = lax.axis_index("s")
    rows = [rows0, rows1]
    fbuf = [fb0, fb1]
    obuf = [out0, out1]
    srcb = [sb0, sb1, sb2, sb3, sb4]
    dstb = [db0, db1, db2, db3, db4]
    gsem = [gs0, gs1]
    fsem = [fs0, fs1]
    ssem = [ss0, ss1]
    isem = [is0, is1, is2, is3, is4]
    dsem = [id0, id1, id2, id3, id4]

    def zrow(i, _):
        for j in range(H // 16):
            out0[i, pl.ds(j * 16, 16)] = jnp.zeros((16,), f32)
        return 0
    lax.fori_loop(0, C, zrow, 0)
    for k in range(RPT // C):
        pltpu.sync_copy(out0, shared.at[pl.ds(s * RPT + k * C, C)])
    plsc.subcore_barrier()

    def edge_loop(f_hbm):
        def block(i, _):
            blk = s * BPT + i
            base0 = blk * K * C

            def ldidx(j):
                sl = j % NIDX
                return (
                    pltpu.async_copy(src.at[pl.ds(base0 + j * C, C)],
                                     srcb[sl], isem[sl]),
                    pltpu.async_copy(dst.at[pl.ds(base0 + j * C, C)],
                                     dstb[sl], dsem[sl]),
                )

            def gat(j):
                return pltpu.async_copy(xw.at[srcb[j % NIDX]], rows[j % 2],
                                        gsem[j % 2])

            def fld(j):
                return pltpu.async_copy(
                    f_hbm.at[pl.ds(base0 + j * C, C)], fbuf[j % 2],
                    fsem[j % 2])

            g = [None] * K
            fd = [None] * K
            sd = [None] * K
            ix = [None] * K
            ix[0] = ldidx(0)
            ix[1] = ldidx(1)
            ix[2] = ldidx(2)
            for j in range(2):
                ix[j][0].wait()
                g[j] = gat(j)
                fd[j] = fld(j)
            for j in range(K):
                b = j % 2
                if j >= 2:
                    sd[j - 2].wait()
                if j + 3 < K:
                    ix[j + 3] = ldidx(j + 3)
                g[j].wait()
                fd[j].wait()

                fb = fbuf[b]
                rw = rows[b]
                ob = obuf[b]

                @plsc.parallel_loop(0, C // 2, step=1, unroll=2)
                def _(pp):
                    for par in range(2):
                        e = 2 * pp + par
                        for q in range(H // 32):
                            xa = rw[e, pl.ds(q * 16, 16)]
                            xb = rw[e, pl.ds(H // 2 + q * 16, 16)]
                            fwv = fb[e, pl.ds(q * 16, 16)]
                            fa = lax.bitcast_convert_type(fwv << 16, f32)
                            fbb = lax.bitcast_convert_type(
                                fwv & (-65536), f32)
                            ob[e, pl.ds(q * 16, 16)] = xa * fa
                            ob[e, pl.ds(H // 2 + q * 16, 16)] = xb * fbb

                ix[j][1].wait()
                sd[j] = pltpu.async_copy(ob, shared.at[dstb[j % NIDX]],
                                         ssem[b], add=True)
                if j + 2 < K:
                    ix[j + 2][0].wait()
                    g[j + 2] = gat(j + 2)
                    fd[j + 2] = fld(j + 2)
            sd[K - 2].wait()
            sd[K - 1].wait()
            return 0
        lax.fori_loop(0, BPT, block, 0)

    @pl.when(c == 0)
    def _():
        edge_loop(f1)

    @pl.when(c == 1)
    def _():
        edge_loop(f2)

    plsc.subcore_barrier()

    @pl.when(c == 0)
    def _():
        pltpu.sync_copy(shared.at[pl.ds(s * RPT, RPT)],
                        agg1.at[pl.ds(s * RPT, RPT)])

    @pl.when(c == 1)
    def _():
        pltpu.sync_copy(shared.at[pl.ds(s * RPT, RPT)],
                        agg2.at[pl.ds(s * RPT, RPT)])


def _sc_call(xw, f1, f2, src2, dst2):
    mesh = plsc.VectorSubcoreMesh(core_axis_name="c", subcore_axis_name="s")
    return pl.kernel(
        _sc_body,
        out_type=[jax.ShapeDtypeStruct((NPAD, H), f32)] * 2,
        mesh=mesh,
        scratch_types=[
            pltpu.VMEM_SHARED((NPAD, H), f32),
            pltpu.VMEM((C, H), f32),
            pltpu.VMEM((C, H), f32),
            pltpu.VMEM((C, H // 2), jnp.int32),
            pltpu.VMEM((C, H // 2), jnp.int32),
            pltpu.VMEM((C, H), f32),
            pltpu.VMEM((C, H), f32),
        ] + [pltpu.VMEM((C,), jnp.int32)] * (2 * NIDX)
        + [pltpu.SemaphoreType.DMA] * (6 + 2 * NIDX),
    )(xw, f1, f2, src2, dst2)


def _tail_body(a1, a2, x1, x2,
               wc1l, bc1l, wc1r, wl1, bl1,
               wc2l, bc2l, wc2r, wl2, bl2,
               wc0a, wc0b, bc0, wca1, bca1, wca2, bca2,
               wl0, bl0, wll1, bll1, wf, bf, out):
    xl1 = x1[...]
    h1 = _mm(a1[...], wc1l[...]) + bc1l[...] + _mm(xl1, wc1r[...])
    h1 = _swish(_mm(h1, wl1[...]) + bl1[...])
    h2 = _mm(a2[...], wc2l[...]) + bc2l[...] + _mm(xl1, wc2r[...])
    h2 = _swish(_mm(h2, wl2[...]) + bl2[...])
    h = _swish(_mm(h1, wc0a[...]) + _mm(h2, wc0b[...]) + bc0[...])
    h = _swish(_mm(h, wca1[...]) + bca1[...])
    h = _swish(_mm(h, wca2[...]) + bca2[...])
    h = h + x2[...]
    h = _swish(_mm(h, wl0[...]) + bl0[...])
    h = _swish(_mm(h, wll1[...]) + bll1[...])
    out[...] = _mm(h, wf[...]) + bf[...]


def _tail_call(a1, a2, x1, x2, weights):
    BN = 2000
    blk = pl.BlockSpec((BN, H), lambda i: (i, 0))
    full = pl.BlockSpec((H, H), lambda i: (0, 0))
    bias = pl.BlockSpec((1, H), lambda i: (0, 0))
    wspecs = []
    for w in weights:
        wspecs.append(bias if w.shape[0] == 1 else full)
    return pl.pallas_call(
        _tail_body,
        grid=(N // BN,),
        in_specs=[blk, blk, blk, blk] + wspecs,
        out_specs=blk,
        out_shape=jax.ShapeDtypeStruct((N, H), f32),
    )(a1, a2, x1, x2, *weights)


def kernel(x, feature1, pos_emb, edge_index, batch, params):
    p = params
    src2 = edge_index[0].astype(jnp.int32)
    dst2 = edge_index[1].astype(jnp.int32)

    def b(name):
        return p[name].reshape(1, H)

    xl1, xl2 = _xlin_call(x, p['W_lin_1'], b('b_lin_1'),
                          p['W_lin_2'], b('b_lin_2'))
    f1, f2 = _feat_call(feature1, pos_emb,
                        p['Wf1_a'], p['Wf1_b'], p['Wf2_a'], p['Wf2_b'])
    agg1, agg2 = _sc_call(xl1, f1, f2, src2, dst2)

    weights = [
        p['Wc1_l'], b('bc1_l'), p['Wc1_r'], p['W_lin1'], b('b_lin1'),
        p['Wc2_l'], b('bc2_l'), p['Wc2_r'], p['W_lin2'], b('b_lin2'),
        p['W_cat0'][:, :H], p['W_cat0'][:, H:], b('b_cat0'),
        p['W_cat1'], b('b_cat1'), p['W_cat2'], b('b_cat2'),
        p['W_l0'], b('b_l0'), p['W_l1'], b('b_l1'),
        p['W_final'], b('b_final'),
    ]
    return _tail_call(agg1, agg2, xl1, xl2, weights)


# speedup vs baseline: 1.0921x; 1.0421x over previous
import functools

import jax
import jax.numpy as jnp
import numpy as np
from jax import lax
from jax.experimental import pallas as pl
from jax.experimental.pallas import tpu as pltpu
from jax.experimental.pallas import tpu_sc as plsc

N = 10000
E = 320000
H = 128
MID = 64
F1 = 12
PE = 16

NC = 2
NS = 16
E2 = E
C = 40
CPT = E2 // C // NS
K = 10
BPT = CPT // K
NPAD = 10240
RPT = NPAD // NS
ZR = 128

f32 = jnp.float32


def _mm(a, w):
    return lax.dot_general(a, w, (((1,), (1,)), ((), ())),
                           preferred_element_type=f32)


def _swish(y):
    return y * jax.nn.sigmoid(y)


def _pack_pairs(y):
    a = y[:, :H // 2].astype(jnp.bfloat16).astype(f32)
    b = y[:, H // 2:].astype(jnp.bfloat16).astype(f32)
    au = lax.bitcast_convert_type(a, jnp.uint32)
    bu = lax.bitcast_convert_type(b, jnp.uint32)
    return lax.bitcast_convert_type((au >> 16) | bu, jnp.int32)


def _xlin_body(x_ref, w1, b1, w2, b2, o1, o2):
    x = x_ref[...]
    y1 = _swish(_mm(x, w1[...]) + b1[...])
    o1[...] = y1
    y2 = _mm(x, w2[...]) + b2[...]
    o2[...] = _swish(y2)


def _xlin_call(x, w1, b1, w2, b2):
    BN = 2000
    full = pl.BlockSpec((H, H), lambda i: (0, 0))
    bias = pl.BlockSpec((1, H), lambda i: (0, 0))
    return pl.pallas_call(
        _xlin_body,
        grid=(N // BN,),
        in_specs=[pl.BlockSpec((BN, H), lambda i: (i, 0)), full, bias, full, bias],
        out_specs=[pl.BlockSpec((BN, H), lambda i: (i, 0))] * 2,
        out_shape=[jax.ShapeDtypeStruct((N, H), f32)] * 2,
    )(x, w1, b1, w2, b2)


def _feat_body(fe_ref, pe_ref, wa1, wb1, wa2, wb2, o1, o2):
    m1 = _mm(wb1[...], wa1[...].T)
    o1[...] = _pack_pairs(_mm(fe_ref[...], m1))
    m2 = _mm(wb2[...], wa2[...].T)
    o2[...] = _pack_pairs(_mm(pe_ref[...], m2))


def _feat_call(feature1, pos_emb, wa1, wb1, wa2, wb2):
    BE = 4000
    ne = feature1.shape[0]
    return pl.pallas_call(
        _feat_body,
        grid=(ne // BE,),
        in_specs=[
            pl.BlockSpec((BE, F1), lambda i: (i, 0)),
            pl.BlockSpec((BE, PE), lambda i: (i, 0)),
            pl.BlockSpec((MID, F1), lambda i: (0, 0)),
            pl.BlockSpec((H, MID), lambda i: (0, 0)),
            pl.BlockSpec((MID, PE), lambda i: (0, 0)),
            pl.BlockSpec((H, MID), lambda i: (0, 0)),
        ],
        out_specs=[pl.BlockSpec((BE, H // 2), lambda i: (i, 0))] * 2,
        out_shape=[jax.ShapeDtypeStruct((ne, H // 2), jnp.int32)] * 2,
    )(feature1, pos_emb, wa1, wb1, wa2, wb2)


NIDX = 5


def _sc_body(xw, f1, f2, src, dst, agg1, agg2,
             shared, rows0, rows1, rows2, fb0, fb1, fb2, out0, out1,
             sb0, sb1, sb2, sb3, sb4, db0, db1, db2, db3, db4,
             gs0, gs1, gs2, fs0, fs1, fs2, ss0, ss1,
             is0, is1, is2, is3, is4, id0, id1, id2, id3, id4):
    c = lax.axis_index("c")
    s = lax.axis_index("s")
    rows = [rows0, rows1, rows2]
    fbuf = [fb0, fb1, fb2]
    obuf = [out0, out1]
    srcb = [sb0, sb1, sb2, sb3, sb4]
    dstb = [db0, db1, db2, db3, db4]
    gsem = [gs0, gs1, gs2]
    fsem = [fs0, fs1, fs2]
    ssem = [ss0, ss1]
    isem = [is0, is1, is2, is3, is4]
    dsem = [id0, id1, id2, id3, id4]

    def zrow(i, _):
        for j in range(H // 16):
            out0[i, pl.ds(j * 16, 16)] = jnp.zeros((16,), f32)
        return 0
    lax.fori_loop(0, C, zrow, 0)
    for k in range(RPT // C):
        pltpu.sync_copy(out0, shared.at[pl.ds(s * RPT + k * C, C)])
    plsc.subcore_barrier()

    def edge_loop(f_hbm):
        def block(i, _):
            blk = s * BPT + i
            base0 = blk * K * C

            def ldidx(j):
                sl = j % NIDX
                return (
                    pltpu.async_copy(src.at[pl.ds(base0 + j * C, C)],
                                     srcb[sl], isem[sl]),
                    pltpu.async_copy(dst.at[pl.ds(base0 + j * C, C)],
                                     dstb[sl], dsem[sl]),
                )

            def gat(j):
                return pltpu.async_copy(xw.at[srcb[j % NIDX]], rows[j % 3],
                                        gsem[j % 3])

            def fld(j):
                return pltpu.async_copy(
                    f_hbm.at[pl.ds(base0 + j * C, C)], fbuf[j % 3],
                    fsem[j % 3])

            g = [None] * K
            fd = [None] * K
            sd = [None] * K
            ix = [None] * K
            ix[0] = ldidx(0)
            ix[1] = ldidx(1)
            ix[2] = ldidx(2)
            for j in range(2):
                ix[j][0].wait()
                g[j] = gat(j)
                fd[j] = fld(j)
            for j in range(K):
                b = j % 2
                if j >= 2:
                    sd[j - 2].wait()
                if j + 3 < K:
                    ix[j + 3] = ldidx(j + 3)
                if j + 2 < K:
                    ix[j + 2][0].wait()
                    g[j + 2] = gat(j + 2)
                    fd[j + 2] = fld(j + 2)
                g[j].wait()
                fd[j].wait()

                fb = fbuf[j % 3]
                rw = rows[j % 3]
                ob = obuf[b]

                @plsc.parallel_loop(0, C // 2, step=1, unroll=2)
                def _(pp):
                    for par in range(2):
                        e = 2 * pp + par
                        for q in range(H // 32):
                            xa = rw[e, pl.ds(q * 16, 16)]
                            xb = rw[e, pl.ds(H // 2 + q * 16, 16)]
                            fwv = fb[e, pl.ds(q * 16, 16)]
                            fa = lax.bitcast_convert_type(fwv << 16, f32)
                            fbb = lax.bitcast_convert_type(
                                fwv & (-65536), f32)
                            ob[e, pl.ds(q * 16, 16)] = xa * fa
                            ob[e, pl.ds(H // 2 + q * 16, 16)] = xb * fbb

                ix[j][1].wait()
                sd[j] = pltpu.async_copy(ob, shared.at[dstb[j % NIDX]],
                                         ssem[b], add=True)
            sd[K - 2].wait()
            sd[K - 1].wait()
            return 0
        lax.fori_loop(0, BPT, block, 0)

    @pl.when(c == 0)
    def _():
        edge_loop(f1)

    @pl.when(c == 1)
    def _():
        edge_loop(f2)

    plsc.subcore_barrier()

    @pl.when(c == 0)
    def _():
        pltpu.sync_copy(shared.at[pl.ds(s * RPT, RPT)],
                        agg1.at[pl.ds(s * RPT, RPT)])

    @pl.when(c == 1)
    def _():
        pltpu.sync_copy(shared.at[pl.ds(s * RPT, RPT)],
                        agg2.at[pl.ds(s * RPT, RPT)])


def _sc_call(xw, f1, f2, src2, dst2):
    mesh = plsc.VectorSubcoreMesh(core_axis_name="c", subcore_axis_name="s")
    return pl.kernel(
        _sc_body,
        out_type=[jax.ShapeDtypeStruct((NPAD, H), f32)] * 2,
        mesh=mesh,
        scratch_types=[
            pltpu.VMEM_SHARED((NPAD, H), f32),
            pltpu.VMEM((C, H), f32),
            pltpu.VMEM((C, H), f32),
            pltpu.VMEM((C, H), f32),
            pltpu.VMEM((C, H // 2), jnp.int32),
            pltpu.VMEM((C, H // 2), jnp.int32),
            pltpu.VMEM((C, H // 2), jnp.int32),
            pltpu.VMEM((C, H), f32),
            pltpu.VMEM((C, H), f32),
        ] + [pltpu.VMEM((C,), jnp.int32)] * (2 * NIDX)
        + [pltpu.SemaphoreType.DMA] * (8 + 2 * NIDX),
    )(xw, f1, f2, src2, dst2)


def _tail_body(a1, a2, x1, x2,
               wc1l, bc1l, wc1r, wl1, bl1,
               wc2l, bc2l, wc2r, wl2, bl2,
               wc0a, wc0b, bc0, wca1, bca1, wca2, bca2,
               wl0, bl0, wll1, bll1, wf, bf, out):
    xl1 = x1[...]
    h1 = _mm(a1[...], wc1l[...]) + bc1l[...] + _mm(xl1, wc1r[...])
    h1 = _swish(_mm(h1, wl1[...]) + bl1[...])
    h2 = _mm(a2[...], wc2l[...]) + bc2l[...] + _mm(xl1, wc2r[...])
    h2 = _swish(_mm(h2, wl2[...]) + bl2[...])
    h = _swish(_mm(h1, wc0a[...]) + _mm(h2, wc0b[...]) + bc0[...])
    h = _swish(_mm(h, wca1[...]) + bca1[...])
    h = _swish(_mm(h, wca2[...]) + bca2[...])
    h = h + x2[...]
    h = _swish(_mm(h, wl0[...]) + bl0[...])
    h = _swish(_mm(h, wll1[...]) + bll1[...])
    out[...] = _mm(h, wf[...]) + bf[...]


def _tail_call(a1, a2, x1, x2, weights):
    BN = 2000
    blk = pl.BlockSpec((BN, H), lambda i: (i, 0))
    full = pl.BlockSpec((H, H), lambda i: (0, 0))
    bias = pl.BlockSpec((1, H), lambda i: (0, 0))
    wspecs = []
    for w in weights:
        wspecs.append(bias if w.shape[0] == 1 else full)
    return pl.pallas_call(
        _tail_body,
        grid=(N // BN,),
        in_specs=[blk, blk, blk, blk] + wspecs,
        out_specs=blk,
        out_shape=jax.ShapeDtypeStruct((N, H), f32),
    )(a1, a2, x1, x2, *weights)


def kernel(x, feature1, pos_emb, edge_index, batch, params):
    p = params
    src2 = edge_index[0].astype(jnp.int32)
    dst2 = edge_index[1].astype(jnp.int32)

    def b(name):
        return p[name].reshape(1, H)

    xl1, xl2 = _xlin_call(x, p['W_lin_1'], b('b_lin_1'),
                          p['W_lin_2'], b('b_lin_2'))
    f1, f2 = _feat_call(feature1, pos_emb,
                        p['Wf1_a'], p['Wf1_b'], p['Wf2_a'], p['Wf2_b'])
    agg1, agg2 = _sc_call(xl1, f1, f2, src2, dst2)

    weights = [
        p['Wc1_l'], b('bc1_l'), p['Wc1_r'], p['W_lin1'], b('b_lin1'),
        p['Wc2_l'], b('bc2_l'), p['Wc2_r'], p['W_lin2'], b('b_lin2'),
        p['W_cat0'][:, :H], p['W_cat0'][:, H:], b('b_cat0'),
        p['W_cat1'], b('b_cat1'), p['W_cat2'], b('b_cat2'),
        p['W_l0'], b('b_l0'), p['W_l1'], b('b_l1'),
        p['W_final'], b('b_final'),
    ]
    return _tail_call(agg1, agg2, xl1, xl2, weights)
